# baseline (speedup 1.0000x reference)
import functools

import jax
import jax.numpy as jnp
from jax import lax
from jax.experimental import pallas as pl
from jax.experimental.pallas import tpu as pltpu
from jax.experimental.pallas import tpu_sc as plsc

B = 16384
K = 64
D = 200
L = 16
NW = 32
BPW = B // NW
G = 64
NG = BPW // G
NV = D // L
TAIL = D - L
GAMMA = 1.0


def kernel(head_index, relation_index, tail_index, entity_embedding,
           relation_embedding):
    head_index = head_index.astype(jnp.int32)
    relation_index = relation_index.astype(jnp.int32)
    tail_index = tail_index.astype(jnp.int32)

    mesh = plsc.VectorSubcoreMesh(core_axis_name="c", subcore_axis_name="s")

    @functools.partial(
        pl.kernel,
        mesh=mesh,
        compiler_params=pltpu.CompilerParams(needs_layout_passes=False,
                                             use_tc_tiling_on_sc=False),
        out_type=jax.ShapeDtypeStruct((B, K), jnp.float32),
        scratch_types=[
            pltpu.VMEM((G,), jnp.int32),
            pltpu.VMEM((G,), jnp.int32),
            pltpu.VMEM((G, K), jnp.int32),
            pltpu.VMEM((G, D), jnp.float32),
            pltpu.VMEM((G, D), jnp.float32),
            pltpu.VMEM((G, D), jnp.float32),
            pltpu.VMEM((K, D), jnp.float32),
            pltpu.VMEM((K, D), jnp.float32),
            pltpu.VMEM((G, K), jnp.float32),
            pltpu.SemaphoreType.DMA,
            pltpu.SemaphoreType.DMA,
            pltpu.SemaphoreType.DMA,
        ],
    )
    def sc_kernel(hidx_hbm, ridx_hbm, tidx_hbm, ent_hbm, rel_hbm, out_hbm,
                  hidx_v, ridx_v, tidx_v, hrows, rrows, hrbuf, tb0, tb1,
                  scores, sem_hr, sem_t0, sem_t1):
        wid = lax.axis_index("s") * 2 + lax.axis_index("c")
        base_w = wid * BPW

        mask_tail = (lax.iota(jnp.int32, L) >= (L - D % L)).astype(jnp.float32)
        lane_last = lax.iota(jnp.int32, L) == (L - 1)

        tbufs = (tb0, tb1)
        tsems = (sem_t0, sem_t1)

        @pl.loop(0, NG)
        def _group(g):
            base = base_w + g * G
            pltpu.sync_copy(hidx_hbm.at[pl.ds(base, G)], hidx_v)
            pltpu.sync_copy(ridx_hbm.at[pl.ds(base, G)], ridx_v)
            pltpu.sync_copy(tidx_hbm.at[pl.ds(base, G)], tidx_v)
            ch = pltpu.async_copy(ent_hbm.at[hidx_v], hrows, sem_hr)
            cr = pltpu.async_copy(rel_hbm.at[ridx_v], rrows, sem_hr)
            pltpu.async_copy(ent_hbm.at[tidx_v.at[0]], tb0, sem_t0)
            pltpu.async_copy(ent_hbm.at[tidx_v.at[1]], tb1, sem_t1)
            ch.wait()
            cr.wait()

            @pl.loop(0, G)
            def _hr(i):
                for j in range(NV):
                    sl = pl.ds(j * L, L)
                    hrbuf[i, sl] = hrows[i, sl] + rrows[i, sl]
                sl = pl.ds(TAIL, L)
                hrbuf[i, sl] = hrows[i, sl] + rrows[i, sl]

            @pl.loop(0, G, step=2)
            def _b2(i0):
                for s in range(2):
                    i = i0 + s
                    tb = tbufs[s]
                    sem = tsems[s]
                    pltpu.make_async_copy(ent_hbm.at[tidx_v.at[i]], tb,
                                          sem).wait()

                    hr = [hrbuf[i, pl.ds(j * L, L)] for j in range(NV)]
                    hrt = hrbuf[i, pl.ds(TAIL, L)]

                    def body(k, carry, tb=tb, hr=hr, hrt=hrt, i=i):
                        acc = jnp.abs(hr[0] - tb[k, pl.ds(0, L)])
                        for j in range(1, NV):
                            acc = acc + jnp.abs(hr[j] - tb[k, pl.ds(j * L, L)])
                        acc = acc + jnp.abs(hrt - tb[k, pl.ds(TAIL, L)]) * mask_tail
                        total = plsc.cumsum(acc)
                        row = jnp.full((L,), i, jnp.int32)
                        col = jnp.full((L,), k, jnp.int32)
                        plsc.store_scatter(scores, [row, col], GAMMA - total,
                                           mask=lane_last)
                        return carry

                    lax.fori_loop(0, K, body, 0)

                    @pl.when(i + 2 < G)
                    def _():
                        pltpu.async_copy(ent_hbm.at[tidx_v.at[i + 2]], tb, sem)

            pltpu.sync_copy(scores, out_hbm.at[pl.ds(base, G)])

    return sc_kernel(head_index, relation_index, tail_index, entity_embedding,
                     relation_embedding)
